# baseline (speedup 1.0000x reference)
import functools

import jax
import jax.numpy as jnp
from jax import lax
from jax.experimental import pallas as pl
from jax.experimental.pallas import tpu as pltpu
from jax.experimental.pallas import tpu_sc as plsc

_L = 16
_RBLK = 128
_NBUF = 6
_PRIME = 4


def _tree_sum(vs):
    while len(vs) > 1:
        vs = [vs[i] + vs[i + 1] for i in range(0, len(vs) - 1, 2)] + (
            [vs[-1]] if len(vs) % 2 else []
        )
    return vs[0]


def _balance_sc(y, aux, slack_arr):
    B, F = y.shape
    info = plsc.get_sparse_core_info()
    nc, ns = info.num_cores, info.num_subcores
    nw = nc * ns
    rows_pw = B // nw
    nblk = rows_pw // _RBLK
    nch = F // _L
    ngrp = _RBLK // _L

    mesh = plsc.VectorSubcoreMesh(core_axis_name="c", subcore_axis_name="s")

    @functools.partial(
        pl.kernel,
        mesh=mesh,
        compiler_params=pltpu.CompilerParams(needs_layout_passes=False),
        out_type=jax.ShapeDtypeStruct((B, F), jnp.float32),
        scratch_types=(
            [pltpu.VMEM((_RBLK, F), jnp.float32) for _ in range(_NBUF)]
            + [
                pltpu.VMEM((12, _L), jnp.float32),
                pltpu.VMEM((_L,), jnp.int32),
                pltpu.VMEM((_L, _L), jnp.float32),
            ]
            + [pltpu.SemaphoreType.DMA for _ in range(2 * _NBUF)]
        ),
    )
    def run(y_hbm, aux_hbm, slk_hbm, out_hbm, *refs):
        bufs = refs[:_NBUF]
        aux_v, slk_v, stage = refs[_NBUF:_NBUF + 3]
        sin = refs[_NBUF + 3:2 * _NBUF + 3]
        sout = refs[2 * _NBUF + 3:]
        wid = lax.axis_index("s") * nc + lax.axis_index("c")
        base = wid * rows_pw

        pltpu.sync_copy(aux_hbm, aux_v)
        pltpu.sync_copy(slk_hbm, slk_v)
        w = [aux_v[k, :] for k in range(nch)]
        cv = aux_v[4, :]
        slk = slk_v[...]
        ii = lax.iota(jnp.int32, _L)

        def copy_in(g):
            return pltpu.make_async_copy(
                y_hbm.at[pl.ds(base + g * _RBLK, _RBLK)], bufs[g % _NBUF], sin[g % _NBUF]
            )

        def copy_out(g):
            return pltpu.make_async_copy(
                bufs[g % _NBUF], out_hbm.at[pl.ds(base + g * _RBLK, _RBLK)], sout[g % _NBUF]
            )

        def compute(buf):
            def group(gr, carry):
                r0 = gr * _L
                for i in range(_L):
                    ys = [buf[r0 + i, pl.ds(k * _L, _L)] for k in range(nch)]
                    p = _tree_sum([ys[k] * w[k] for k in range(nch)] + [cv])
                    stage[i, :] = p
                cols = [
                    plsc.load_gather(stage, [ii, jnp.full((_L,), l, jnp.int32)])
                    for l in range(_L)
                ]
                d = _tree_sum(cols)
                rows = ii + r0
                cur = plsc.load_gather(buf, [rows, slk])
                plsc.store_scatter(buf, [rows, slk], cur + d)
                return carry

            lax.fori_loop(0, ngrp, group, 0)

        for b in range(min(_PRIME, nblk)):
            copy_in(b).start()

        for g in range(nblk):
            copy_in(g).wait()
            compute(bufs[g % _NBUF])
            copy_out(g).start()
            nxt = g + _PRIME
            if nxt < nblk:
                if nxt >= _NBUF:
                    copy_out(nxt - _NBUF).wait()
                copy_in(nxt).start()

        for g in range(max(nblk - _NBUF, 0), nblk):
            copy_out(g).wait()

    return run(y, aux, slack_arr)


def kernel(y, means, stds, asset_idx, liability_idx, equity_idx, slack_idx):
    f32 = jnp.float32
    B, F = y.shape
    sign = (
        jnp.zeros((F,), f32)
        .at[asset_idx].set(1.0)
        .at[liability_idx].set(-1.0)
        .at[equity_idx].set(-1.0)
    )
    inv = 1.0 / stds[slack_idx]
    w = sign * stds * inv
    c = jnp.sum(sign * means) * inv
    aux = jnp.zeros((12, _L), f32)
    aux = aux.at[0:4].set(w.reshape(4, _L))
    aux = aux.at[4, 0].set(c)
    slack_arr = jnp.full((_L,), slack_idx, jnp.int32)
    return _balance_sc(y.astype(f32), aux, slack_arr)
